# Initial kernel scaffold; baseline (speedup 1.0000x reference)
import jax
import jax.numpy as jnp
from jax.experimental import pallas as pl


def kernel(x, ids, cond, sid, quant_table, cond_table, ch_table, sub_table):
    raise NotImplementedError("write your pallas kernel here")


# speedup vs baseline: 2.9543x; 2.9543x over previous
import jax
import jax.numpy as jnp
from jax import lax
from jax.experimental import pallas as pl
from jax.experimental.pallas import tpu as pltpu
from jax.experimental.pallas import tpu_sc as plsc

B, C, T, D = 8, 64, 1024, 64
QL, NCLS, NCH, NSUB = 256, 1000, 64, 1000
NC, NS = 2, 16
NW = NC * NS
CHUNK = 128

_MESH = dict(core_axis_name="c", subcore_axis_name="s", num_cores=NC,
             num_subcores=NS)


def _phase1_body(ids_hbm, sid_hbm, cond_hbm, quant_hbm, condz_hbm, ch_hbm,
                 sub_hbm, qc_out, cese_out,
                 idsv, sidv, qt, che_all, sub_all, qcbuf, cidx, cebuf):
    w = lax.axis_index("s") * NC + lax.axis_index("c")
    iota = jnp.arange(16, dtype=jnp.int32)

    pltpu.sync_copy(ids_hbm, idsv)
    pltpu.sync_copy(quant_hbm, qt)
    pltpu.sync_copy(ch_hbm.at[idsv], che_all)
    for cc in range(2):
        c = w * 2 + cc
        che_v = [che_all[c, pl.ds(16 * k, 16)] for k in range(4)]

        def row_body(r, carry, _che_v=che_v):
            for k in range(4):
                qcbuf[r, pl.ds(16 * k, 16)] = (qt[r, pl.ds(16 * k, 16)]
                                               + _che_v[k])
            return carry

        lax.fori_loop(0, QL, row_body, 0)
        pltpu.sync_copy(qcbuf, qc_out.at[c])

    b = w // 4
    t0 = (w % 4) * 256
    pltpu.sync_copy(sid_hbm, sidv)
    pltpu.sync_copy(sub_hbm.at[sidv], sub_all)
    se_v = [sub_all[b, pl.ds(16 * k, 16)] for k in range(4)]
    for j in range(2):
        tj = t0 + j * CHUNK
        pltpu.sync_copy(cond_hbm.at[b, pl.ds(tj, CHUNK)], cidx)
        pltpu.sync_copy(condz_hbm.at[cidx], cebuf)

        def row2_body(r, carry):
            for k in range(4):
                plsc.addupdate(cebuf.at[r, pl.ds(16 * k, 16)], se_v[k])
            return carry

        lax.fori_loop(0, CHUNK, row2_body, 0)
        pltpu.sync_copy(cebuf, cese_out.at[b, pl.ds(tj, CHUNK)])


def _phase2_body(x_hbm, qc_hbm, cese_hbm, out_hbm, xidx, buf1, buf2):
    w = lax.axis_index("s") * NC + lax.axis_index("c")
    b = w // 4
    c0 = (w % 4) * 16

    def tj_body(j, carry):
        t0 = j * CHUNK
        pltpu.sync_copy(cese_hbm.at[b, pl.ds(t0, CHUNK)], buf2)

        def c_body(i, carry2):
            c = c0 + i
            r = b * C + c
            pltpu.sync_copy(x_hbm.at[b, c, pl.ds(t0, CHUNK)], xidx)
            pltpu.sync_copy(qc_hbm.at[c].at[xidx], buf1)

            def add_body(rr, carry3):
                for k in range(4):
                    plsc.addupdate(buf1.at[rr, pl.ds(16 * k, 16)],
                                   buf2[rr, pl.ds(16 * k, 16)])
                return carry3

            lax.fori_loop(0, CHUNK, add_body, 0)
            pltpu.sync_copy(buf1, out_hbm.at[r, pl.ds(t0, CHUNK)])
            return carry2

        lax.fori_loop(0, 16, c_body, 0)
        return carry

    lax.fori_loop(0, T // CHUNK, tj_body, 0)


def kernel(x, ids, cond, sid, quant_table, cond_table, ch_table, sub_table):
    x32 = x.astype(jnp.int32)
    ids32 = ids.astype(jnp.int32)
    cond32 = cond.reshape(B, T).astype(jnp.int32)
    sid32 = jnp.pad(sid.reshape(B).astype(jnp.int32), (0, 8))
    condz = cond_table.at[0].set(0.0)

    p1 = pl.kernel(
        _phase1_body,
        out_type=[jax.ShapeDtypeStruct((C, QL, D), jnp.float32),
                  jax.ShapeDtypeStruct((B, T, D), jnp.float32)],
        mesh=plsc.VectorSubcoreMesh(**_MESH),
        compiler_params=pltpu.CompilerParams(use_tc_tiling_on_sc=False),
        scratch_types=[
            pltpu.VMEM((NCH,), jnp.int32),
            pltpu.VMEM((16,), jnp.int32),
            pltpu.VMEM((QL, D), jnp.float32),
            pltpu.VMEM((NCH, D), jnp.float32),
            pltpu.VMEM((16, D), jnp.float32),
            pltpu.VMEM((QL, D), jnp.float32),
            pltpu.VMEM((CHUNK,), jnp.int32),
            pltpu.VMEM((CHUNK, D), jnp.float32),
        ],
    )
    qc, cese = p1(ids32, sid32, cond32, quant_table, condz, ch_table,
                  sub_table)

    p2 = pl.kernel(
        _phase2_body,
        out_type=jax.ShapeDtypeStruct((B * C, T, D), jnp.float32),
        mesh=plsc.VectorSubcoreMesh(**_MESH),
        compiler_params=pltpu.CompilerParams(use_tc_tiling_on_sc=False),
        scratch_types=[
            pltpu.VMEM((CHUNK,), jnp.int32),
            pltpu.VMEM((CHUNK, D), jnp.float32),
            pltpu.VMEM((CHUNK, D), jnp.float32),
        ],
    )
    return p2(x32, qc, cese)


# speedup vs baseline: 3.1170x; 1.0551x over previous
import jax
import jax.numpy as jnp
from jax import lax
from jax.experimental import pallas as pl
from jax.experimental.pallas import tpu as pltpu
from jax.experimental.pallas import tpu_sc as plsc

B, C, T, D = 8, 64, 1024, 64
QL, NCLS, NCH, NSUB = 256, 1000, 64, 1000
NC, NS = 2, 16
NW = NC * NS
CHUNK = 128

_MESH = dict(core_axis_name="c", subcore_axis_name="s", num_cores=NC,
             num_subcores=NS)


def _phase1_body(ids_hbm, sid_hbm, cond_hbm, quant_hbm, condz_hbm, ch_hbm,
                 sub_hbm, qc_out, cese_out,
                 idsv, sidv, qt, che_all, sub_all, qcbuf, cidx, cebuf):
    w = lax.axis_index("s") * NC + lax.axis_index("c")
    iota = jnp.arange(16, dtype=jnp.int32)

    pltpu.sync_copy(ids_hbm, idsv)
    pltpu.sync_copy(quant_hbm, qt)
    pltpu.sync_copy(ch_hbm.at[idsv], che_all)
    for cc in range(2):
        c = w * 2 + cc
        che_v = [che_all[c, pl.ds(16 * k, 16)] for k in range(4)]

        def row_body(r, carry, _che_v=che_v):
            for k in range(4):
                qcbuf[r, pl.ds(16 * k, 16)] = (qt[r, pl.ds(16 * k, 16)]
                                               + _che_v[k])
            return carry

        lax.fori_loop(0, QL, row_body, 0)
        pltpu.sync_copy(qcbuf, qc_out.at[c])

    b = w // 4
    t0 = (w % 4) * 256
    pltpu.sync_copy(sid_hbm, sidv)
    pltpu.sync_copy(sub_hbm.at[sidv], sub_all)
    se_v = [sub_all[b, pl.ds(16 * k, 16)] for k in range(4)]
    for j in range(2):
        tj = t0 + j * CHUNK
        pltpu.sync_copy(cond_hbm.at[b, pl.ds(tj, CHUNK)], cidx)
        pltpu.sync_copy(condz_hbm.at[cidx], cebuf)

        def row2_body(r, carry):
            for k in range(4):
                plsc.addupdate(cebuf.at[r, pl.ds(16 * k, 16)], se_v[k])
            return carry

        lax.fori_loop(0, CHUNK, row2_body, 0)
        pltpu.sync_copy(cebuf, cese_out.at[b, pl.ds(tj, CHUNK)])


def _phase2_body(x_hbm, qc_hbm, cese_hbm, out_hbm, xbuf, buf1, buf2,
                 gsem, wsem):
    w = lax.axis_index("s") * NC + lax.axis_index("c")
    b = w // 4
    c0 = (w % 4) * 16

    def gather(i, p, t0):
        return pltpu.make_async_copy(
            qc_hbm.at[c0 + i].at[xbuf.at[i]], buf1.at[p], gsem.at[p])

    def write(i, p, t0):
        return pltpu.make_async_copy(
            buf1.at[p], out_hbm.at[b * C + c0 + i, pl.ds(t0, CHUNK)],
            wsem.at[p])

    def tj_body(j, carry):
        t0 = j * CHUNK
        pltpu.sync_copy(x_hbm.at[b, pl.ds(c0, 16), pl.ds(t0, CHUNK)], xbuf)
        pltpu.sync_copy(cese_hbm.at[b, pl.ds(t0, CHUNK)], buf2)
        gather(0, 0, t0).start()

        def c_body(i, carry2):
            p = i & 1

            @pl.when(i < 15)
            def _prefetch():
                @pl.when(i >= 1)
                def _():
                    write(i - 1, 1 - p, t0).wait()
                gather(i + 1, 1 - p, t0).start()

            gather(i, p, t0).wait()

            def add_body(rr, carry3):
                for u in range(2):
                    for k in range(4):
                        plsc.addupdate(
                            buf1.at[p, 2 * rr + u, pl.ds(16 * k, 16)],
                            buf2[2 * rr + u, pl.ds(16 * k, 16)])
                return carry3

            lax.fori_loop(0, CHUNK // 2, add_body, 0)
            write(i, p, t0).start()
            return carry2

        lax.fori_loop(0, 16, c_body, 0)
        write(14, 0, t0).wait()
        write(15, 1, t0).wait()
        return carry

    lax.fori_loop(0, T // CHUNK, tj_body, 0)


def kernel(x, ids, cond, sid, quant_table, cond_table, ch_table, sub_table):
    x32 = x.astype(jnp.int32)
    ids32 = ids.astype(jnp.int32)
    cond32 = cond.reshape(B, T).astype(jnp.int32)
    sid32 = jnp.pad(sid.reshape(B).astype(jnp.int32), (0, 8))
    condz = cond_table.at[0].set(0.0)

    p1 = pl.kernel(
        _phase1_body,
        out_type=[jax.ShapeDtypeStruct((C, QL, D), jnp.float32),
                  jax.ShapeDtypeStruct((B, T, D), jnp.float32)],
        mesh=plsc.VectorSubcoreMesh(**_MESH),
        compiler_params=pltpu.CompilerParams(use_tc_tiling_on_sc=False),
        scratch_types=[
            pltpu.VMEM((NCH,), jnp.int32),
            pltpu.VMEM((16,), jnp.int32),
            pltpu.VMEM((QL, D), jnp.float32),
            pltpu.VMEM((NCH, D), jnp.float32),
            pltpu.VMEM((16, D), jnp.float32),
            pltpu.VMEM((QL, D), jnp.float32),
            pltpu.VMEM((CHUNK,), jnp.int32),
            pltpu.VMEM((CHUNK, D), jnp.float32),
        ],
    )
    qc, cese = p1(ids32, sid32, cond32, quant_table, condz, ch_table,
                  sub_table)

    p2 = pl.kernel(
        _phase2_body,
        out_type=jax.ShapeDtypeStruct((B * C, T, D), jnp.float32),
        mesh=plsc.VectorSubcoreMesh(**_MESH),
        compiler_params=pltpu.CompilerParams(use_tc_tiling_on_sc=False),
        scratch_types=[
            pltpu.VMEM((16, CHUNK), jnp.int32),
            pltpu.VMEM((2, CHUNK, D), jnp.float32),
            pltpu.VMEM((CHUNK, D), jnp.float32),
            pltpu.SemaphoreType.DMA((2,)),
            pltpu.SemaphoreType.DMA((2,)),
        ],
    )
    return p2(x32, qc, cese)
